# baseline (speedup 1.0000x reference)
import jax
import jax.numpy as jnp
from jax import lax
from jax.experimental import pallas as pl
from jax.experimental.pallas import tpu as pltpu
from jax.experimental.pallas import tpu_sc as plsc

N_FIELDS = 26
VOCAB_P1 = 100001
EMB = 16
BATCH = 16384

NC, NS, LANES = 2, 16, 16
NW = NC * NS
PLANES = N_FIELDS * EMB
P_PER_W = PLANES // NW
OUT_CHUNK = 4096
N_OUT_CHUNKS = BATCH // OUT_CHUNK
G_PER_CHUNK = OUT_CHUNK // LANES


def _body(tab_hbm, idx_hbm, out_hbm, plane_v, idx_v, out_v, sem, sem_out):
    wid = lax.axis_index("s") * NC + lax.axis_index("c")
    p0 = wid * P_PER_W

    def plane_body(k, _):
        p = p0 + k
        f = p // EMB
        e = lax.rem(p, EMB)

        @pl.when(jnp.logical_or(k == 0, e == 0))
        def _():
            pltpu.async_copy(idx_hbm.at[f], idx_v, sem)

        plane_copy = pltpu.async_copy(tab_hbm.at[f, e], plane_v, sem)

        @pl.when(jnp.logical_or(k == 0, e == 0))
        def _():
            pltpu.make_async_copy(idx_hbm.at[f], idx_v, sem).wait()

        plane_copy.wait()

        @pl.when(k > 0)
        def _():
            for _i in range(N_OUT_CHUNKS):
                pltpu.make_async_copy(
                    out_hbm.at[f, e, pl.ds(0, OUT_CHUNK)], out_v.at[0], sem_out
                ).wait()

        for q in range(N_OUT_CHUNKS):
            b = q % 2
            qb = q * OUT_CHUNK

            @plsc.parallel_loop(0, G_PER_CHUNK, unroll=16)
            def gather_body(i):
                out_v[b, pl.ds(i * LANES, LANES)] = plsc.load_gather(
                    plane_v, [idx_v[pl.ds(qb + i * LANES, LANES)]]
                )

            pltpu.async_copy(
                out_v.at[b], out_hbm.at[f, e, pl.ds(qb, OUT_CHUNK)], sem_out
            )
        return 0

    lax.fori_loop(0, P_PER_W, plane_body, 0)

    for _i in range(N_OUT_CHUNKS):
        pltpu.make_async_copy(
            out_hbm.at[0, 0, pl.ds(0, OUT_CHUNK)], out_v.at[0], sem_out
        ).wait()


@jax.jit
def _embed(indices, tables):
    tab_t = jnp.transpose(tables, (0, 2, 1))
    idx_t = indices.T
    grid_kernel = pl.kernel(
        _body,
        out_type=jax.ShapeDtypeStruct((N_FIELDS, EMB, BATCH), jnp.float32),
        mesh=plsc.VectorSubcoreMesh(core_axis_name="c", subcore_axis_name="s"),
        scratch_types=[
            pltpu.VMEM((VOCAB_P1,), jnp.float32),
            pltpu.VMEM((BATCH,), jnp.int32),
            pltpu.VMEM((2, OUT_CHUNK), jnp.float32),
            pltpu.SemaphoreType.DMA,
            pltpu.SemaphoreType.DMA,
        ],
        compiler_params=pltpu.CompilerParams(needs_layout_passes=False),
    )
    out_t = grid_kernel(tab_t, idx_t)
    return jnp.transpose(out_t, (2, 0, 1))


def kernel(indices, tables):
    return _embed(indices, tables)


# speedup vs baseline: 1.0052x; 1.0052x over previous
import jax
import jax.numpy as jnp
from jax import lax
from jax.experimental import pallas as pl
from jax.experimental.pallas import tpu as pltpu
from jax.experimental.pallas import tpu_sc as plsc

N_FIELDS = 26
VOCAB_P1 = 100001
EMB = 16
BATCH = 16384

NC, NS, LANES = 2, 16, 16
NW = NC * NS
PLANES = N_FIELDS * EMB
P_PER_W = PLANES // NW
OUT_CHUNK = 4096
N_OUT_CHUNKS = BATCH // OUT_CHUNK
G_PER_CHUNK = OUT_CHUNK // LANES


def _body(tab_hbm, idx_hbm, out_hbm, plane_v, idx_v, out_v, sem, sem_out):
    wid = lax.axis_index("s") * NC + lax.axis_index("c")
    p0 = wid * P_PER_W

    def plane_body(k, _):
        p = p0 + k
        f = p // EMB
        e = lax.rem(p, EMB)

        @pl.when(jnp.logical_or(k == 0, e == 0))
        def _():
            pltpu.sync_copy(idx_hbm.at[f], idx_v)

        pltpu.sync_copy(tab_hbm.at[f, e], plane_v)

        @pl.when(k > 0)
        def _():
            for _i in range(N_OUT_CHUNKS):
                pltpu.make_async_copy(
                    out_hbm.at[f, e, pl.ds(0, OUT_CHUNK)], out_v.at[0], sem_out
                ).wait()

        for q in range(N_OUT_CHUNKS):
            b = q % 2
            qb = q * OUT_CHUNK

            @plsc.parallel_loop(0, G_PER_CHUNK, unroll=8)
            def gather_body(i):
                out_v[b, pl.ds(i * LANES, LANES)] = plsc.load_gather(
                    plane_v, [idx_v[pl.ds(qb + i * LANES, LANES)]]
                )

            pltpu.async_copy(
                out_v.at[b], out_hbm.at[f, e, pl.ds(qb, OUT_CHUNK)], sem_out
            )
        return 0

    lax.fori_loop(0, P_PER_W, plane_body, 0)

    for _i in range(N_OUT_CHUNKS):
        pltpu.make_async_copy(
            out_hbm.at[0, 0, pl.ds(0, OUT_CHUNK)], out_v.at[0], sem_out
        ).wait()


@jax.jit
def _embed(indices, tables):
    tab_t = jnp.transpose(tables, (0, 2, 1))
    idx_t = indices.T
    grid_kernel = pl.kernel(
        _body,
        out_type=jax.ShapeDtypeStruct((N_FIELDS, EMB, BATCH), jnp.float32),
        mesh=plsc.VectorSubcoreMesh(core_axis_name="c", subcore_axis_name="s"),
        scratch_types=[
            pltpu.VMEM((VOCAB_P1,), jnp.float32),
            pltpu.VMEM((BATCH,), jnp.int32),
            pltpu.VMEM((2, OUT_CHUNK), jnp.float32),
            pltpu.SemaphoreType.DMA,
            pltpu.SemaphoreType.DMA,
        ],
        compiler_params=pltpu.CompilerParams(needs_layout_passes=False),
    )
    out_t = grid_kernel(tab_t, idx_t)
    return jnp.transpose(out_t, (2, 0, 1))


def kernel(indices, tables):
    return _embed(indices, tables)
